# Initial kernel scaffold; baseline (speedup 1.0000x reference)
import jax
import jax.numpy as jnp
from jax.experimental import pallas as pl


def kernel(user_indices, pos_item_indices, neg_item_indices, E0, adj_src, adj_dst, adj_vals):
    raise NotImplementedError("write your pallas kernel here")


# speedup vs baseline: 11.3345x; 11.3345x over previous
import functools

import jax
import jax.numpy as jnp
from jax import lax
from jax.experimental import pallas as pl
from jax.experimental.pallas import tpu as pltpu
from jax.experimental.pallas import tpu_sc as plsc

N_USERS = 50000
N_ITEMS = 50000
EMBED = 32
NE_HALF = 800000
BATCH = 4096

NC = 2
NS = 16
SUB = 128
SPB = 4
BLK = SUB * SPB
BPT = 100
EPT = BLK * BPT
E_PAD = EPT * NS
HALF_PAD = 50048
N_PAD = 2 * HALF_PAD
RPT = HALF_PAD // NS
WB_CHUNKS = tuple((o, min(BLK, RPT - o)) for o in range(0, RPT, BLK))

_MESH = plsc.VectorSubcoreMesh(
    core_axis_name="c", subcore_axis_name="s", num_cores=NC, num_subcores=NS)


@functools.partial(
    pl.kernel,
    out_type=jax.ShapeDtypeStruct((N_PAD, EMBED), jnp.float32),
    mesh=_MESH,
    compiler_params=pltpu.CompilerParams(use_tc_tiling_on_sc=False),
    scratch_types=[
        pltpu.VMEM((SPB, SUB), jnp.int32),
        pltpu.VMEM((SPB, SUB), jnp.int32),
        pltpu.VMEM((SPB, SUB), jnp.float32),
        pltpu.VMEM((BLK, EMBED), jnp.float32),
        pltpu.VMEM_SHARED((HALF_PAD, EMBED), jnp.float32),
        pltpu.SemaphoreType.DMA,
    ],
)
def _layer(e_hbm, src_hbm, dst_hbm, vals_hbm, out_hbm,
           src_v, dst_v, vals_v, rows_v, acc, sem):
    c = lax.axis_index("c")
    s = lax.axis_index("s")
    zero16 = jnp.zeros((16,), jnp.float32)

    def _zero_body(i, carry):
        rows_v[i, 0:16] = zero16
        rows_v[i, 16:32] = zero16
        return carry
    lax.fori_loop(0, min(BLK, RPT), _zero_body, 0)
    for off, sz in WB_CHUNKS:
        pltpu.sync_copy(rows_v.at[pl.ds(0, sz)],
                        acc.at[pl.ds(s * RPT + off, sz)])
    plsc.subcore_barrier()

    def _block_body(b, carry):
        g = (s * BPT + b) * SPB
        pltpu.sync_copy(src_hbm.at[c, pl.ds(g, SPB)], src_v)
        pltpu.sync_copy(dst_hbm.at[c, pl.ds(g, SPB)], dst_v)
        pltpu.sync_copy(vals_hbm.at[c, pl.ds(g, SPB)], vals_v)
        copies = [
            pltpu.async_copy(e_hbm.at[src_v.at[j]],
                             rows_v.at[pl.ds(j * SUB, SUB)], sem)
            for j in range(SPB)
        ]
        for cp in copies:
            cp.wait()
        for j in range(SPB):
            def _scale_body(k16, carry2, j=j):
                vv = vals_v[j, pl.ds(k16 * 16, 16)]
                for u in range(16):
                    v = vv[u]
                    r = j * SUB + k16 * 16 + u
                    rows_v[r, 0:16] = rows_v[r, 0:16] * v
                    rows_v[r, 16:32] = rows_v[r, 16:32] * v
                return carry2
            lax.fori_loop(0, SUB // 16, _scale_body, 0)
            pltpu.sync_copy(rows_v.at[pl.ds(j * SUB, SUB)],
                            acc.at[dst_v.at[j]], add=True)
        return carry
    lax.fori_loop(0, BPT, _block_body, 0)

    plsc.subcore_barrier()
    base = (1 - c) * HALF_PAD + s * RPT
    for off, sz in WB_CHUNKS:
        pltpu.sync_copy(acc.at[pl.ds(s * RPT + off, sz)],
                        rows_v.at[pl.ds(0, sz)])
        pltpu.sync_copy(rows_v.at[pl.ds(0, sz)],
                        out_hbm.at[pl.ds(base + off, sz)])


_GPT = BATCH // (NC * NS)


@functools.partial(
    pl.kernel,
    out_type=(jax.ShapeDtypeStruct((BATCH, EMBED), jnp.float32),) * 3,
    mesh=_MESH,
    compiler_params=pltpu.CompilerParams(use_tc_tiling_on_sc=False),
    scratch_types=[
        pltpu.VMEM((_GPT,), jnp.int32),
        pltpu.VMEM((_GPT, EMBED), jnp.float32),
        pltpu.SemaphoreType.DMA,
    ],
)
def _final(e0_hbm, e1_hbm, e2_hbm, e3_hbm, ui_hbm, pi_hbm, ni_hbm,
           ou_hbm, op_hbm, on_hbm, idx_v, buf, sem):
    c = lax.axis_index("c")
    s = lax.axis_index("s")
    base = (c * NS + s) * _GPT
    zero16 = jnp.zeros((16,), jnp.float32)
    quarter = jnp.float32(0.25)
    for idx_hbm, out_hbm in ((ui_hbm, ou_hbm), (pi_hbm, op_hbm),
                             (ni_hbm, on_hbm)):
        def _zero_body(i, carry):
            buf[i, 0:16] = zero16
            buf[i, 16:32] = zero16
            return carry
        lax.fori_loop(0, _GPT, _zero_body, 0)
        pltpu.sync_copy(idx_hbm.at[pl.ds(base, _GPT)], idx_v)
        copies = [pltpu.async_copy(t.at[idx_v], buf, sem, add=True)
                  for t in (e0_hbm, e1_hbm, e2_hbm, e3_hbm)]
        for cp in copies:
            cp.wait()
        def _scale_body(i, carry):
            buf[i, 0:16] = buf[i, 0:16] * quarter
            buf[i, 16:32] = buf[i, 16:32] * quarter
            return carry
        lax.fori_loop(0, _GPT, _scale_body, 0)
        pltpu.sync_copy(buf, out_hbm.at[pl.ds(base, _GPT)])


def _prep_edges(adj_src, adj_dst, adj_vals):
    pad = E_PAD - NE_HALF
    zi = jnp.zeros((pad,), jnp.int32)
    zf = jnp.zeros((pad,), jnp.float32)
    src32 = adj_src.astype(jnp.int32)
    dst32 = adj_dst.astype(jnp.int32)
    s0 = jnp.concatenate([src32[:NE_HALF], zi])
    d0 = jnp.concatenate([dst32[:NE_HALF] - N_USERS, zi])
    s1 = jnp.concatenate([src32[NE_HALF:] + (HALF_PAD - N_USERS), zi])
    d1 = jnp.concatenate([dst32[NE_HALF:], zi])
    v0 = jnp.concatenate([adj_vals[:NE_HALF], zf])
    v1 = jnp.concatenate([adj_vals[NE_HALF:], zf])
    src = jnp.stack([s0, s1]).reshape(NC, E_PAD // SUB, SUB)
    dst = jnp.stack([d0, d1]).reshape(NC, E_PAD // SUB, SUB)
    vals = jnp.stack([v0, v1]).reshape(NC, E_PAD // SUB, SUB)
    return src, dst, vals


def _pad_table(E0):
    z = jnp.zeros((HALF_PAD - N_USERS, EMBED), jnp.float32)
    return jnp.concatenate([E0[:N_USERS], z, E0[N_USERS:], z], axis=0)


def kernel(user_indices, pos_item_indices, neg_item_indices, E0,
           adj_src, adj_dst, adj_vals):
    src, dst, vals = _prep_edges(adj_src, adj_dst, adj_vals)
    e0 = _pad_table(E0.astype(jnp.float32))
    e1 = _layer(e0, src, dst, vals)
    e2 = _layer(e1, src, dst, vals)
    e3 = _layer(e2, src, dst, vals)
    ui = user_indices.astype(jnp.int32)
    pi = pos_item_indices.astype(jnp.int32) + HALF_PAD
    ni = neg_item_indices.astype(jnp.int32) + HALF_PAD
    return _final(e0, e1, e2, e3, ui, pi, ni)


# speedup vs baseline: 13.4576x; 1.1873x over previous
import functools

import jax
import jax.numpy as jnp
from jax import lax
from jax.experimental import pallas as pl
from jax.experimental.pallas import tpu as pltpu
from jax.experimental.pallas import tpu_sc as plsc

N_USERS = 50000
N_ITEMS = 50000
EMBED = 32
NE_HALF = 800000
BATCH = 4096

NC = 2
NS = 16
SUB = 128
SPB = 2
BLK = SUB * SPB
BPT = 200
EPT = BLK * BPT
E_PAD = EPT * NS
HALF_PAD = 50048
N_PAD = 2 * HALF_PAD
RPT = HALF_PAD // NS
WB_CHUNKS = tuple((o, min(BLK, RPT - o)) for o in range(0, RPT, BLK))

_MESH = plsc.VectorSubcoreMesh(
    core_axis_name="c", subcore_axis_name="s", num_cores=NC, num_subcores=NS)


@functools.partial(
    pl.kernel,
    out_type=jax.ShapeDtypeStruct((N_PAD, EMBED), jnp.float32),
    mesh=_MESH,
    compiler_params=pltpu.CompilerParams(use_tc_tiling_on_sc=False),
    scratch_types=[
        (pltpu.VMEM((SPB, SUB), jnp.int32),) * 2,
        (pltpu.VMEM((SPB, SUB), jnp.int32),) * 2,
        (pltpu.VMEM((SPB, SUB), jnp.float32),) * 2,
        (pltpu.VMEM((BLK, EMBED), jnp.float32),) * 2,
        pltpu.VMEM_SHARED((HALF_PAD, EMBED), jnp.float32),
        pltpu.SemaphoreType.DMA,
        pltpu.SemaphoreType.DMA,
        pltpu.SemaphoreType.DMA,
    ],
)
def _layer(e_hbm, src_hbm, dst_hbm, vals_hbm, out_hbm,
           src_v, dst_v, vals_v, rows_v, acc, gsem, ssem, isem):
    c = lax.axis_index("c")
    s = lax.axis_index("s")
    zero16 = jnp.zeros((16,), jnp.float32)

    def idx_base(b):
        return (s * BPT + b) * SPB

    def fire_sv(b, p):
        g = idx_base(b)
        pltpu.async_copy(src_hbm.at[c, pl.ds(g, SPB)], src_v[p], isem)
        pltpu.async_copy(vals_hbm.at[c, pl.ds(g, SPB)], vals_v[p], isem)

    def fire_d(b, p):
        pltpu.async_copy(dst_hbm.at[c, pl.ds(idx_base(b), SPB)],
                         dst_v[p], isem)

    def wait_isem(n):
        for _ in range(n):
            pltpu.make_async_copy(
                src_hbm.at[c, pl.ds(0, SPB)], src_v[0], isem).wait()

    def fire_gathers(p):
        for j in range(SPB):
            pltpu.async_copy(e_hbm.at[src_v[p].at[j]],
                             rows_v[p].at[pl.ds(j * SUB, SUB)], gsem)

    def wait_gathers(p):
        for j in range(SPB):
            pltpu.make_async_copy(e_hbm.at[src_v[p].at[j]],
                                  rows_v[p].at[pl.ds(j * SUB, SUB)],
                                  gsem).wait()

    def fire_scatters(p):
        for j in range(SPB):
            pltpu.async_copy(rows_v[p].at[pl.ds(j * SUB, SUB)],
                             acc.at[dst_v[p].at[j]], ssem, add=True)

    def wait_scatters(p):
        for j in range(SPB):
            pltpu.make_async_copy(rows_v[p].at[pl.ds(j * SUB, SUB)],
                                  acc.at[dst_v[p].at[j]], ssem).wait()

    def scale(p):
        for j in range(SPB):
            def _scale_body(k16, carry, j=j):
                vv = vals_v[p][j, pl.ds(k16 * 16, 16)]
                for u in range(16):
                    v = vv[u]
                    r = j * SUB + k16 * 16 + u
                    rows_v[p][r, 0:16] = rows_v[p][r, 0:16] * v
                    rows_v[p][r, 16:32] = rows_v[p][r, 16:32] * v
                return carry
            lax.fori_loop(0, SUB // 16, _scale_body, 0)

    def _zero_body(i, carry):
        rows_v[0][i, 0:16] = zero16
        rows_v[0][i, 16:32] = zero16
        return carry
    lax.fori_loop(0, BLK, _zero_body, 0)
    zcopies = [
        pltpu.async_copy(rows_v[0].at[pl.ds(0, sz)],
                         acc.at[pl.ds(s * RPT + off, sz)], gsem)
        for off, sz in WB_CHUNKS
    ]
    for cp in zcopies:
        cp.wait()
    plsc.subcore_barrier()

    def section(b, p, first2, prev_none, nxt):
        fire_d(b, p)
        if nxt:
            fire_sv(b + 1, 1 - p)
        wait_gathers(p)
        scale(p)
        wait_isem(3 if nxt else 1)
        if nxt:
            fire_gathers(1 - p)
        for j in range(SPB):
            pltpu.sync_copy(rows_v[p].at[pl.ds(j * SUB, SUB)],
                            acc.at[dst_v[p].at[j]], add=True)

    fire_sv(0, 0)
    wait_isem(2)
    fire_gathers(0)
    section(0, 0, first2=True, prev_none=True, nxt=True)
    section(1, 1, first2=True, prev_none=False, nxt=True)

    def _pair_body(i, carry):
        section(2 * i, 0, first2=False, prev_none=False, nxt=True)
        section(2 * i + 1, 1, first2=False, prev_none=False, nxt=True)
        return carry
    lax.fori_loop(1, BPT // 2 - 1, _pair_body, 0)

    section(BPT - 2, 0, first2=False, prev_none=False, nxt=True)
    section(BPT - 1, 1, first2=False, prev_none=False, nxt=False)

    plsc.subcore_barrier()
    base = (1 - c) * HALF_PAD + s * RPT
    nwb = len(WB_CHUNKS)

    def rd(k):
        off, sz = WB_CHUNKS[k]
        return (acc.at[pl.ds(s * RPT + off, sz)],
                rows_v[k & 1].at[pl.ds(0, sz)])

    def wr(k):
        off, sz = WB_CHUNKS[k]
        return (rows_v[k & 1].at[pl.ds(0, sz)],
                out_hbm.at[pl.ds(base + off, sz)])

    pltpu.async_copy(*rd(0), gsem)
    for k in range(nwb):
        pltpu.make_async_copy(*rd(k), gsem).wait()
        if k + 1 < nwb:
            if k >= 1:
                pltpu.make_async_copy(*wr(k - 1), ssem).wait()
            pltpu.async_copy(*rd(k + 1), gsem)
        pltpu.async_copy(*wr(k), ssem)
    pltpu.make_async_copy(*wr(nwb - 2), ssem).wait()
    pltpu.make_async_copy(*wr(nwb - 1), ssem).wait()


_GPT = BATCH // (NC * NS)


@functools.partial(
    pl.kernel,
    out_type=(jax.ShapeDtypeStruct((BATCH, EMBED), jnp.float32),) * 3,
    mesh=_MESH,
    compiler_params=pltpu.CompilerParams(use_tc_tiling_on_sc=False),
    scratch_types=[
        pltpu.VMEM((_GPT,), jnp.int32),
        pltpu.VMEM((_GPT, EMBED), jnp.float32),
        pltpu.SemaphoreType.DMA,
    ],
)
def _final(e0_hbm, e1_hbm, e2_hbm, e3_hbm, ui_hbm, pi_hbm, ni_hbm,
           ou_hbm, op_hbm, on_hbm, idx_v, buf, sem):
    c = lax.axis_index("c")
    s = lax.axis_index("s")
    base = (c * NS + s) * _GPT
    zero16 = jnp.zeros((16,), jnp.float32)
    quarter = jnp.float32(0.25)
    for idx_hbm, out_hbm in ((ui_hbm, ou_hbm), (pi_hbm, op_hbm),
                             (ni_hbm, on_hbm)):
        def _zero_body(i, carry):
            buf[i, 0:16] = zero16
            buf[i, 16:32] = zero16
            return carry
        lax.fori_loop(0, _GPT, _zero_body, 0)
        pltpu.sync_copy(idx_hbm.at[pl.ds(base, _GPT)], idx_v)
        copies = [pltpu.async_copy(t.at[idx_v], buf, sem, add=True)
                  for t in (e0_hbm, e1_hbm, e2_hbm, e3_hbm)]
        for cp in copies:
            cp.wait()
        def _scale_body(i, carry):
            buf[i, 0:16] = buf[i, 0:16] * quarter
            buf[i, 16:32] = buf[i, 16:32] * quarter
            return carry
        lax.fori_loop(0, _GPT, _scale_body, 0)
        pltpu.sync_copy(buf, out_hbm.at[pl.ds(base, _GPT)])


def _prep_edges(adj_src, adj_dst, adj_vals):
    pad = E_PAD - NE_HALF
    zi = jnp.zeros((pad,), jnp.int32)
    zf = jnp.zeros((pad,), jnp.float32)
    src32 = adj_src.astype(jnp.int32)
    dst32 = adj_dst.astype(jnp.int32)
    s0 = jnp.concatenate([src32[:NE_HALF], zi])
    d0 = jnp.concatenate([dst32[:NE_HALF] - N_USERS, zi])
    s1 = jnp.concatenate([src32[NE_HALF:] + (HALF_PAD - N_USERS), zi])
    d1 = jnp.concatenate([dst32[NE_HALF:], zi])
    v0 = jnp.concatenate([adj_vals[:NE_HALF], zf])
    v1 = jnp.concatenate([adj_vals[NE_HALF:], zf])
    src = jnp.stack([s0, s1]).reshape(NC, E_PAD // SUB, SUB)
    dst = jnp.stack([d0, d1]).reshape(NC, E_PAD // SUB, SUB)
    vals = jnp.stack([v0, v1]).reshape(NC, E_PAD // SUB, SUB)
    return src, dst, vals


def _pad_table(E0):
    z = jnp.zeros((HALF_PAD - N_USERS, EMBED), jnp.float32)
    return jnp.concatenate([E0[:N_USERS], z, E0[N_USERS:], z], axis=0)


def kernel(user_indices, pos_item_indices, neg_item_indices, E0,
           adj_src, adj_dst, adj_vals):
    src, dst, vals = _prep_edges(adj_src, adj_dst, adj_vals)
    e0 = _pad_table(E0.astype(jnp.float32))
    e1 = _layer(e0, src, dst, vals)
    e2 = _layer(e1, src, dst, vals)
    e3 = _layer(e2, src, dst, vals)
    ui = user_indices.astype(jnp.int32)
    pi = pos_item_indices.astype(jnp.int32) + HALF_PAD
    ni = neg_item_indices.astype(jnp.int32) + HALF_PAD
    return _final(e0, e1, e2, e3, ui, pi, ni)
